# baseline (speedup 1.0000x reference)
import functools

import jax
import jax.numpy as jnp
from jax.experimental import pallas as pl
from jax.experimental.pallas import tpu as pltpu

_C = 96
_H = 224
_NC = 512
_PS = 864
_L = _H * _H

_RT = 1792
_LT = 1792


def _unfold_body(xp_ref, o_ref):
    xp = xp_ref[0]
    for ki in range(3):
        for kj in range(3):
            o_ref[0, ki * 3 + kj] = xp[ki:ki + _H, kj:kj + _H]


def _proto_body(scal_ref, z_ref, c_ref, o_ref):
    tempv = scal_ref[0]
    alpha = scal_ref[1]
    beta = scal_ref[2]
    f = z_ref[...]
    c = c_ref[...]
    fb = f.astype(jnp.bfloat16)
    cb = c.astype(jnp.bfloat16)
    g = jax.lax.dot_general(fb, cb, (((1,), (1,)), ((), ())),
                            preferred_element_type=jnp.float32)
    f2 = jnp.sum(f * f, axis=1, keepdims=True)
    c2 = jnp.sum(c * c, axis=1)[None, :]
    d2 = f2 + c2 - 2.0 * g
    d = jnp.sqrt(jnp.maximum(d2, 1e-12))
    neg = -d * tempv
    m = jnp.max(neg, axis=1, keepdims=True)
    e = jnp.exp(neg - m)
    s = (e / jnp.sum(e, axis=1, keepdims=True)).astype(jnp.bfloat16)
    t = jax.lax.dot_general(s, cb, (((1,), (0,)), ((), ())),
                            preferred_element_type=jnp.float32)
    o_ref[...] = alpha * t + beta * f


def _conv_body(z_ref, w_ref, b_ref, o_ref):
    i = pl.program_id(0)
    z = z_ref[...]
    ch = jax.lax.broadcasted_iota(jnp.int32, z.shape, 0)
    l = jax.lax.broadcasted_iota(jnp.int32, z.shape, 1) + i * _LT
    top = ((ch % 9) < 3) & (l < _H)
    left = ((ch % 3) == 0) & ((l % _H) == 0)
    zm = jnp.where(top | left, 0.0, z).astype(jnp.bfloat16)
    wb = w_ref[...].astype(jnp.bfloat16)
    o_ref[...] = jax.lax.dot_general(wb, zm, (((1,), (0,)), ((), ())),
                                     preferred_element_type=jnp.float32) + b_ref[...]


def kernel(x, weight, bias, cluster_centers, temp):
    xp = jnp.pad(x[0], ((0, 0), (1, 1), (1, 1)))

    z4 = pl.pallas_call(
        _unfold_body,
        grid=(_C,),
        in_specs=[pl.BlockSpec((1, _H + 2, _H + 2), lambda i: (i, 0, 0))],
        out_specs=pl.BlockSpec((1, 9, _H, _H), lambda i: (i, 0, 0, 0)),
        out_shape=jax.ShapeDtypeStruct((_C, 9, _H, _H), jnp.float32),
        compiler_params=pltpu.CompilerParams(
            dimension_semantics=("parallel",)),
    )(xp)
    zf = z4.reshape(_L, _PS)

    tempf = jnp.asarray(temp, jnp.float32)
    scal = jnp.stack([tempf, tempf / (tempf + 1.0), 1.0 / (tempf + 1.0), tempf])

    f2 = pl.pallas_call(
        _proto_body,
        grid=(_L // _RT,),
        in_specs=[
            pl.BlockSpec(memory_space=pltpu.SMEM),
            pl.BlockSpec((_RT, _PS), lambda i: (i, 0)),
            pl.BlockSpec((_NC, _PS), lambda i: (0, 0)),
        ],
        out_specs=pl.BlockSpec((_RT, _PS), lambda i: (i, 0)),
        out_shape=jax.ShapeDtypeStruct((_L, _PS), jnp.float32),
        compiler_params=pltpu.CompilerParams(
            dimension_semantics=("parallel",)),
    )(scal, zf, cluster_centers)

    z2 = f2.reshape(_PS, _L)
    w2 = weight.reshape(_C, _PS)
    b2 = bias.reshape(_C, 1)

    out = pl.pallas_call(
        _conv_body,
        grid=(_L // _LT,),
        in_specs=[
            pl.BlockSpec((_PS, _LT), lambda i: (0, i)),
            pl.BlockSpec((_C, _PS), lambda i: (0, 0)),
            pl.BlockSpec((_C, 1), lambda i: (0, 0)),
        ],
        out_specs=pl.BlockSpec((_C, _LT), lambda i: (0, i)),
        out_shape=jax.ShapeDtypeStruct((_C, _L), jnp.float32),
        compiler_params=pltpu.CompilerParams(
            dimension_semantics=("parallel",)),
    )(z2, w2, b2)

    return out.reshape(1, _C, _H, _H)


# speedup vs baseline: 1.1486x; 1.1486x over previous
import jax
import jax.numpy as jnp
from jax.experimental import pallas as pl
from jax.experimental.pallas import tpu as pltpu

_C = 96
_H = 224
_NC = 512
_PS = 864
_L = _H * _H

_RT = 1792
_LT = 1792


def _unfold_body(x_ref, o_ref):
    xp = jnp.pad(x_ref[0], ((1, 1), (1, 1))).astype(jnp.bfloat16)
    for ki in range(3):
        for kj in range(3):
            o_ref[0, ki * 3 + kj] = xp[ki:ki + _H, kj:kj + _H]


def _proto_body(scal_ref, z_ref, c_ref, o_ref):
    tempv = scal_ref[0]
    alpha = scal_ref[1]
    beta = scal_ref[2]
    fb = z_ref[...]
    f = fb.astype(jnp.float32)
    c = c_ref[...]
    cb = c.astype(jnp.bfloat16)
    g = jax.lax.dot_general(fb, cb, (((1,), (1,)), ((), ())),
                            preferred_element_type=jnp.float32)
    f2 = jnp.sum(f * f, axis=1, keepdims=True)
    c2 = jnp.sum(c * c, axis=1)[None, :]
    d2 = f2 + c2 - 2.0 * g
    d = jnp.sqrt(jnp.maximum(d2, 1e-12))
    neg = -d * tempv
    m = jnp.max(neg, axis=1, keepdims=True)
    e = jnp.exp(neg - m)
    s = (e / jnp.sum(e, axis=1, keepdims=True)).astype(jnp.bfloat16)
    t = jax.lax.dot_general(s, cb, (((1,), (0,)), ((), ())),
                            preferred_element_type=jnp.float32)
    o_ref[...] = (alpha * t + beta * f).astype(jnp.bfloat16)


def _conv_body(z_ref, w_ref, b_ref, o_ref):
    i = pl.program_id(0)
    z = z_ref[...]
    ch = jax.lax.broadcasted_iota(jnp.int32, z.shape, 0)
    l = jax.lax.broadcasted_iota(jnp.int32, z.shape, 1) + i * _LT
    top = ((ch % 9) < 3) & (l < _H)
    left = ((ch % 3) == 0) & ((l % _H) == 0)
    zm = jnp.where(top | left, jnp.bfloat16(0), z)
    wb = w_ref[...].astype(jnp.bfloat16)
    o_ref[...] = jax.lax.dot_general(wb, zm, (((1,), (0,)), ((), ())),
                                     preferred_element_type=jnp.float32) + b_ref[...]


def kernel(x, weight, bias, cluster_centers, temp):
    z4 = pl.pallas_call(
        _unfold_body,
        grid=(_C,),
        in_specs=[pl.BlockSpec((1, _H, _H), lambda i: (i, 0, 0))],
        out_specs=pl.BlockSpec((1, 9, _H, _H), lambda i: (i, 0, 0, 0)),
        out_shape=jax.ShapeDtypeStruct((_C, 9, _H, _H), jnp.bfloat16),
        compiler_params=pltpu.CompilerParams(
            dimension_semantics=("parallel",)),
    )(x[0])
    zf = z4.reshape(_L, _PS)

    tempf = jnp.asarray(temp, jnp.float32)
    scal = jnp.stack([tempf, tempf / (tempf + 1.0), 1.0 / (tempf + 1.0), tempf])

    f2 = pl.pallas_call(
        _proto_body,
        grid=(_L // _RT,),
        in_specs=[
            pl.BlockSpec(memory_space=pltpu.SMEM),
            pl.BlockSpec((_RT, _PS), lambda i: (i, 0)),
            pl.BlockSpec((_NC, _PS), lambda i: (0, 0)),
        ],
        out_specs=pl.BlockSpec((_RT, _PS), lambda i: (i, 0)),
        out_shape=jax.ShapeDtypeStruct((_L, _PS), jnp.bfloat16),
        compiler_params=pltpu.CompilerParams(
            dimension_semantics=("parallel",)),
    )(scal, zf, cluster_centers)

    z2 = f2.reshape(_PS, _L)
    w2 = weight.reshape(_C, _PS)
    b2 = bias.reshape(_C, 1)

    out = pl.pallas_call(
        _conv_body,
        grid=(_L // _LT,),
        in_specs=[
            pl.BlockSpec((_PS, _LT), lambda i: (0, i)),
            pl.BlockSpec((_C, _PS), lambda i: (0, 0)),
            pl.BlockSpec((_C, 1), lambda i: (0, 0)),
        ],
        out_specs=pl.BlockSpec((_C, _LT), lambda i: (0, i)),
        out_shape=jax.ShapeDtypeStruct((_C, _L), jnp.float32),
        compiler_params=pltpu.CompilerParams(
            dimension_semantics=("parallel",)),
    )(z2, w2, b2)

    return out.reshape(1, _C, _H, _H)


# speedup vs baseline: 1.2519x; 1.0900x over previous
import jax
import jax.numpy as jnp
from jax.experimental import pallas as pl
from jax.experimental.pallas import tpu as pltpu

_C = 96
_H = 224
_NC = 512
_PS = 864
_L = _H * _H

_RT = 1792
_LT = 1792


def _unfold_body(x_ref, o_ref):
    xp = jnp.pad(x_ref[0], ((1, 1), (1, 1))).astype(jnp.bfloat16)
    for ki in range(3):
        for kj in range(3):
            o_ref[0, ki * 3 + kj] = xp[ki:ki + _H, kj:kj + _H]


def _proto_body(scal_ref, z_ref, cn_ref, c_ref, c2_ref, o_ref):
    tempv = scal_ref[0]
    fb = z_ref[...]
    g = jax.lax.dot_general(fb, cn_ref[...], (((1,), (1,)), ((), ())),
                            preferred_element_type=jnp.float32)
    f2 = jnp.sum(fb.astype(jnp.float32) ** 2, axis=1, keepdims=True)
    d2 = (g + f2) + c2_ref[...]
    d = jnp.sqrt(jnp.maximum(d2, 1e-12))
    neg = d * (-tempv)
    m = jnp.max(neg, axis=1, keepdims=True)
    eb = jnp.exp((neg - m).astype(jnp.bfloat16))
    ssum = jnp.sum(eb.astype(jnp.float32), axis=1, keepdims=True)
    rr = (tempv / ssum).astype(jnp.bfloat16)
    s = eb * rr
    t = jax.lax.dot_general(s, c_ref[...], (((1,), (0,)), ((), ())),
                            preferred_element_type=jnp.float32)
    o_ref[...] = t.astype(jnp.bfloat16) + fb


def _conv_body(z_ref, k_ref, w_ref, b_ref, o_ref):
    zm = z_ref[...] * k_ref[0]
    o_ref[...] = jax.lax.dot_general(w_ref[...], zm, (((1,), (0,)), ((), ())),
                                     preferred_element_type=jnp.float32) + b_ref[...]


def _keep_planes():
    ch = jnp.arange(_PS).reshape(_PS, 1)
    col = jnp.arange(_LT).reshape(1, _LT)
    top = ((ch % 9) < 3) & (col < _H)
    left = ((ch % 3) == 0) & ((col % _H) == 0)
    k0 = jnp.where(top | left, 0.0, 1.0)
    k1 = jnp.where(left, 0.0, 1.0)
    return jnp.stack([k0, k1]).astype(jnp.bfloat16)


def kernel(x, weight, bias, cluster_centers, temp):
    z4 = pl.pallas_call(
        _unfold_body,
        grid=(_C,),
        in_specs=[pl.BlockSpec((1, _H, _H), lambda i: (i, 0, 0))],
        out_specs=pl.BlockSpec((1, 9, _H, _H), lambda i: (i, 0, 0, 0)),
        out_shape=jax.ShapeDtypeStruct((_C, 9, _H, _H), jnp.bfloat16),
        compiler_params=pltpu.CompilerParams(
            dimension_semantics=("parallel",)),
    )(x[0])
    zf = z4.reshape(_L, _PS)

    tempf = jnp.asarray(temp, jnp.float32)
    scal = jnp.stack([tempf, tempf, tempf, tempf])
    cn = (-2.0 * cluster_centers).astype(jnp.bfloat16)
    cb = cluster_centers.astype(jnp.bfloat16)
    c2 = jnp.sum(cluster_centers * cluster_centers, axis=1).reshape(1, _NC)

    f2 = pl.pallas_call(
        _proto_body,
        grid=(_L // _RT,),
        in_specs=[
            pl.BlockSpec(memory_space=pltpu.SMEM),
            pl.BlockSpec((_RT, _PS), lambda i: (i, 0)),
            pl.BlockSpec((_NC, _PS), lambda i: (0, 0)),
            pl.BlockSpec((_NC, _PS), lambda i: (0, 0)),
            pl.BlockSpec((1, _NC), lambda i: (0, 0)),
        ],
        out_specs=pl.BlockSpec((_RT, _PS), lambda i: (i, 0)),
        out_shape=jax.ShapeDtypeStruct((_L, _PS), jnp.bfloat16),
        compiler_params=pltpu.CompilerParams(
            dimension_semantics=("parallel",)),
    )(scal, zf, cn, cb, c2)

    z2 = f2.reshape(_PS, _L)
    w2 = (weight.reshape(_C, _PS) / (tempf + 1.0)).astype(jnp.bfloat16)
    b2 = bias.reshape(_C, 1)

    out = pl.pallas_call(
        _conv_body,
        grid=(_L // _LT,),
        in_specs=[
            pl.BlockSpec((_PS, _LT), lambda i: (0, i)),
            pl.BlockSpec((1, _PS, _LT), lambda i: (jnp.minimum(i, 1), 0, 0)),
            pl.BlockSpec((_C, _PS), lambda i: (0, 0)),
            pl.BlockSpec((_C, 1), lambda i: (0, 0)),
        ],
        out_specs=pl.BlockSpec((_C, _LT), lambda i: (0, i)),
        out_shape=jax.ShapeDtypeStruct((_C, _L), jnp.float32),
        compiler_params=pltpu.CompilerParams(
            dimension_semantics=("parallel",)),
    )(z2, _keep_planes(), w2, b2)

    return out.reshape(1, _C, _H, _H)


# speedup vs baseline: 1.3369x; 1.0679x over previous
import jax
import jax.numpy as jnp
from jax.experimental import pallas as pl
from jax.experimental.pallas import tpu as pltpu

_C = 96
_H = 224
_NC = 512
_PS = 864
_L = _H * _H

_RT = 3584
_LT = 1792


def _unfold_body(x_ref, o_ref):
    for c in range(4):
        xp = jnp.pad(x_ref[c], ((1, 1), (1, 1))).astype(jnp.bfloat16)
        for ki in range(3):
            for kj in range(3):
                o_ref[c, ki * 3 + kj] = xp[ki:ki + _H, kj:kj + _H]


def _proto_body(scal_ref, z_ref, cn_ref, c_ref, c2_ref, o_ref):
    tempv = scal_ref[0]
    fb = z_ref[...]
    g = jax.lax.dot_general(fb, cn_ref[...], (((1,), (1,)), ((), ())),
                            preferred_element_type=jnp.float32)
    f2 = jnp.sum(fb * fb, axis=1, keepdims=True, dtype=jnp.float32)
    d2 = jnp.maximum((g + f2) + c2_ref[...], 1e-12)
    d = d2 * jax.lax.rsqrt(d2)
    neg = d * (-tempv)
    m = jnp.max(neg, axis=1, keepdims=True)
    eb = jnp.exp((neg - m).astype(jnp.bfloat16))
    ssum = jnp.sum(eb.astype(jnp.float32), axis=1, keepdims=True)
    rr = (tempv / ssum).astype(jnp.bfloat16)
    s = eb * rr
    t = jax.lax.dot_general(s, c_ref[...], (((1,), (0,)), ((), ())),
                            preferred_element_type=jnp.float32)
    o_ref[...] = t.astype(jnp.bfloat16) + fb


def _conv_body(z_ref, k_ref, w_ref, b_ref, o_ref):
    zm = z_ref[...] * k_ref[0]
    o_ref[...] = jax.lax.dot_general(w_ref[...], zm, (((1,), (0,)), ((), ())),
                                     preferred_element_type=jnp.float32) + b_ref[...]


def _keep_planes():
    ch = jnp.arange(_PS).reshape(_PS, 1)
    col = jnp.arange(_LT).reshape(1, _LT)
    top = ((ch % 9) < 3) & (col < _H)
    left = ((ch % 3) == 0) & ((col % _H) == 0)
    k0 = jnp.where(top | left, 0.0, 1.0)
    k1 = jnp.where(left, 0.0, 1.0)
    return jnp.stack([k0, k1]).astype(jnp.bfloat16)


def kernel(x, weight, bias, cluster_centers, temp):
    z4 = pl.pallas_call(
        _unfold_body,
        grid=(_C // 4,),
        in_specs=[pl.BlockSpec((4, _H, _H), lambda i: (i, 0, 0))],
        out_specs=pl.BlockSpec((4, 9, _H, _H), lambda i: (i, 0, 0, 0)),
        out_shape=jax.ShapeDtypeStruct((_C, 9, _H, _H), jnp.bfloat16),
        compiler_params=pltpu.CompilerParams(
            dimension_semantics=("parallel",)),
    )(x[0])
    zf = z4.reshape(_L, _PS)

    tempf = jnp.asarray(temp, jnp.float32)
    scal = jnp.stack([tempf, tempf, tempf, tempf])
    cn = (-2.0 * cluster_centers).astype(jnp.bfloat16)
    cb = cluster_centers.astype(jnp.bfloat16)
    c2 = jnp.sum(cluster_centers * cluster_centers, axis=1).reshape(1, _NC)

    f2 = pl.pallas_call(
        _proto_body,
        grid=(_L // _RT,),
        in_specs=[
            pl.BlockSpec(memory_space=pltpu.SMEM),
            pl.BlockSpec((_RT, _PS), lambda i: (i, 0)),
            pl.BlockSpec((_NC, _PS), lambda i: (0, 0)),
            pl.BlockSpec((_NC, _PS), lambda i: (0, 0)),
            pl.BlockSpec((1, _NC), lambda i: (0, 0)),
        ],
        out_specs=pl.BlockSpec((_RT, _PS), lambda i: (i, 0)),
        out_shape=jax.ShapeDtypeStruct((_L, _PS), jnp.bfloat16),
        compiler_params=pltpu.CompilerParams(
            dimension_semantics=("parallel",)),
    )(scal, zf, cn, cb, c2)

    z2 = f2.reshape(_PS, _L)
    w2 = (weight.reshape(_C, _PS) / (tempf + 1.0)).astype(jnp.bfloat16)
    b2 = bias.reshape(_C, 1)

    out = pl.pallas_call(
        _conv_body,
        grid=(_L // _LT,),
        in_specs=[
            pl.BlockSpec((_PS, _LT), lambda i: (0, i)),
            pl.BlockSpec((1, _PS, _LT), lambda i: (jnp.minimum(i, 1), 0, 0)),
            pl.BlockSpec((_C, _PS), lambda i: (0, 0)),
            pl.BlockSpec((_C, 1), lambda i: (0, 0)),
        ],
        out_specs=pl.BlockSpec((_C, _LT), lambda i: (0, i)),
        out_shape=jax.ShapeDtypeStruct((_C, _L), jnp.float32),
        compiler_params=pltpu.CompilerParams(
            dimension_semantics=("parallel",)),
    )(z2, _keep_planes(), w2, b2)

    return out.reshape(1, _C, _H, _H)
